# baseline (speedup 1.0000x reference)
import jax
import jax.numpy as jnp
from jax import lax
from jax.experimental import pallas as pl

_K = 512
_E = 8
_W = 128 * _E
_R = 128
_LO0 = -2139095041
_HI0 = 2139095041

_DOT = jax.lax.Precision.HIGHEST


def _unsort(m):
    return jnp.where(m >= 0, m, m ^ jnp.int32(0x7FFFFFFF))


def _topk_mask_body(x_ref, o_ref):
    x = x_ref[0]
    lane_e = lax.broadcasted_iota(jnp.int32, (_E, _W), 1) % _E
    ee = lax.broadcasted_iota(jnp.int32, (_E, _W), 0)
    et = (lane_e == ee).astype(jnp.float32)
    lo0 = jnp.full((1, _E), _LO0, jnp.int32)
    hi0 = jnp.full((1, _E), _HI0, jnp.int32)

    def body(_, carry):
        lo, hi = carry
        mid = (lo & hi) + ((lo ^ hi) >> 1)
        midf = lax.bitcast_convert_type(_unsort(mid), jnp.float32)
        mid_lane = jnp.dot(midf, et, precision=_DOT)
        maskf = (x >= mid_lane).astype(jnp.float32)
        per_e = jnp.dot(maskf, et.T, precision=_DOT)
        cnt = jnp.sum(per_e, axis=0, keepdims=True)
        ge = cnt >= _K
        return jnp.where(ge, mid, lo), jnp.where(ge, hi, mid)

    lo, _ = lax.fori_loop(0, 32, body, (lo0, hi0))
    tf = lax.bitcast_convert_type(_unsort(lo), jnp.float32)
    tf_lane = jnp.dot(tf, et, precision=_DOT)
    o_ref[0] = (x >= tf_lane).astype(jnp.float32)


def kernel(scores):
    bsz, nmax, _, ens = scores.shape
    s = scores.reshape(bsz, nmax, nmax * ens)
    out = pl.pallas_call(
        _topk_mask_body,
        grid=(bsz,),
        in_specs=[pl.BlockSpec((1, _R, _W), lambda b: (b, 0, 0))],
        out_specs=pl.BlockSpec((1, _R, _W), lambda b: (b, 0, 0)),
        out_shape=jax.ShapeDtypeStruct(s.shape, jnp.float32),
    )(s)
    return out.reshape(scores.shape)


# speedup vs baseline: 8.2516x; 8.2516x over previous
import jax
import jax.numpy as jnp
from jax import lax
from jax.experimental import pallas as pl

_K = 512
_N = 16384
_ROWS = 32
_LO0 = -2139095041
_HI0 = 2139095041


def _unsort(m):
    return jnp.where(m >= 0, m, m ^ jnp.int32(0x7FFFFFFF))


def _topk_mask_body(x_ref, o_ref):
    x = x_ref[...]
    r = x.shape[0]
    lo0 = jnp.full((r, 1), _LO0, jnp.int32)
    hi0 = jnp.full((r, 1), _HI0, jnp.int32)

    def body(_, carry):
        lo, hi = carry
        mid = (lo & hi) + ((lo ^ hi) >> 1)
        midf = lax.bitcast_convert_type(_unsort(mid), jnp.float32)
        cnt = jnp.sum((x >= midf).astype(jnp.int32), axis=1, keepdims=True)
        ge = cnt >= _K
        return jnp.where(ge, mid, lo), jnp.where(ge, hi, mid)

    lo, _ = lax.fori_loop(0, 32, body, (lo0, hi0))
    tf = lax.bitcast_convert_type(_unsort(lo), jnp.float32)
    o_ref[...] = (x >= tf).astype(jnp.float32)


def kernel(scores):
    bsz, nmax, _, ens = scores.shape
    s = jnp.transpose(scores, (0, 3, 1, 2)).reshape(bsz * ens, nmax * nmax)
    out = pl.pallas_call(
        _topk_mask_body,
        grid=(s.shape[0] // _ROWS,),
        in_specs=[pl.BlockSpec((_ROWS, _N), lambda r: (r, 0))],
        out_specs=pl.BlockSpec((_ROWS, _N), lambda r: (r, 0)),
        out_shape=jax.ShapeDtypeStruct(s.shape, jnp.float32),
    )(s)
    out = out.reshape(bsz, ens, nmax, nmax)
    return jnp.transpose(out, (0, 2, 3, 1))
